# Initial kernel scaffold; baseline (speedup 1.0000x reference)
import jax
import jax.numpy as jnp
from jax.experimental import pallas as pl


def kernel(x, a_hat, w_pre, b_pre, w_mean, b_mean, w_proj, b_proj, bn_gamma, bn_beta, bn_rmean, bn_rvar):
    raise NotImplementedError("write your pallas kernel here")


# speedup vs baseline: 3.5280x; 3.5280x over previous
import functools
import math

import jax
import jax.numpy as jnp
from jax.experimental import pallas as pl
from jax.experimental.pallas import tpu as pltpu

_LANE = 128
_VMEM_LIMIT = 48 * 1024 * 1024


def _ceil_to(n, m):
    return ((n + m - 1) // m) * m


def _mlp_kernel(x_ref, wpre_ref, bpre_ref, wmean_ref, bmean_ref, m_ref):
    h = jnp.maximum(
        jnp.dot(x_ref[...], wpre_ref[...], preferred_element_type=jnp.float32)
        + bpre_ref[...], 0.0)
    m_ref[...] = (jnp.dot(h, wmean_ref[...], preferred_element_type=jnp.float32)
                  + bmean_ref[...])


def _prop_kernel(a_ref, z_ref, m_ref, o_ref, *, tm):
    i = pl.program_id(0)
    a16 = a_ref[...].astype(jnp.bfloat16)
    zh16 = (0.5 * z_ref[...]).astype(jnp.bfloat16)
    acc = jnp.dot(a16, zh16, preferred_element_type=jnp.float32)
    o_ref[...] = acc + 0.5 * m_ref[pl.ds(i * tm, tm), :]


def _prop_project_kernel(a_ref, z_ref, m_ref, wp_ref, bp_ref, o_ref, *, tm):
    i = pl.program_id(0)
    a16 = a_ref[...].astype(jnp.bfloat16)
    zh16 = (0.5 * z_ref[...]).astype(jnp.bfloat16)
    z2 = (jnp.dot(a16, zh16, preferred_element_type=jnp.float32)
          + 0.5 * m_ref[pl.ds(i * tm, tm), :])
    o_ref[...] = (jnp.dot(z2, wp_ref[...], preferred_element_type=jnp.float32)
                  + bp_ref[...])


def kernel(x, a_hat, w_pre, b_pre, w_mean, b_mean, w_proj, b_proj,
           bn_gamma, bn_beta, bn_rmean, bn_rvar):
    n, fin = x.shape
    hid = w_pre.shape[1]
    out_ch = w_proj.shape[1]
    f32 = jnp.float32

    fp = _ceil_to(fin, _LANE)
    hp = _ceil_to(hid, _LANE)
    op = _ceil_to(out_ch, _LANE)

    tm = 256
    n_pad = _ceil_to(max(n, tm), tm)
    grid_r = n_pad // tm

    def pad2(arr, r, c):
        if arr.shape == (r, c):
            return arr
        return jnp.pad(arr, ((0, r - arr.shape[0]), (0, c - arr.shape[1])))

    a_p = pad2(a_hat, n_pad, n_pad)
    x_p = pad2(x.astype(f32), n_pad, fp)
    w_pre_p = pad2(w_pre, fp, hp)
    b_pre_p = pad2(b_pre, 1, hp)
    w_mean_p = pad2(w_mean, hp, op)
    b_mean_p = pad2(b_mean, 1, op)

    eps = 1e-5
    scale = bn_gamma[0] * jax.lax.rsqrt(bn_rvar[0] + eps)
    shift = bn_beta[0] - bn_rmean[0] * scale
    w_proj_f = pad2(scale[:, None] * w_proj, op, op)
    b_proj_f = pad2(b_proj + (shift @ w_proj)[None, :], 1, op)

    cparams = pltpu.CompilerParams(
        dimension_semantics=("parallel",), vmem_limit_bytes=_VMEM_LIMIT)

    m = pl.pallas_call(
        _mlp_kernel,
        out_shape=jax.ShapeDtypeStruct((n_pad, op), f32),
        grid=(grid_r,),
        in_specs=[
            pl.BlockSpec((tm, fp), lambda i: (i, 0)),
            pl.BlockSpec((fp, hp), lambda i: (0, 0)),
            pl.BlockSpec((1, hp), lambda i: (0, 0)),
            pl.BlockSpec((hp, op), lambda i: (0, 0)),
            pl.BlockSpec((1, op), lambda i: (0, 0)),
        ],
        out_specs=pl.BlockSpec((tm, op), lambda i: (i, 0)),
        compiler_params=cparams,
    )(x_p, w_pre_p, b_pre_p, w_mean_p, b_mean_p)

    z1 = pl.pallas_call(
        functools.partial(_prop_kernel, tm=tm),
        out_shape=jax.ShapeDtypeStruct((n_pad, op), f32),
        grid=(grid_r,),
        in_specs=[
            pl.BlockSpec((tm, n_pad), lambda i: (i, 0)),
            pl.BlockSpec((n_pad, op), lambda i: (0, 0)),
            pl.BlockSpec((n_pad, op), lambda i: (0, 0)),
        ],
        out_specs=pl.BlockSpec((tm, op), lambda i: (i, 0)),
        compiler_params=cparams,
    )(a_p, m, m)

    out_p = pl.pallas_call(
        functools.partial(_prop_project_kernel, tm=tm),
        out_shape=jax.ShapeDtypeStruct((n_pad, op), f32),
        grid=(grid_r,),
        in_specs=[
            pl.BlockSpec((tm, n_pad), lambda i: (i, 0)),
            pl.BlockSpec((n_pad, op), lambda i: (0, 0)),
            pl.BlockSpec((n_pad, op), lambda i: (0, 0)),
            pl.BlockSpec((op, op), lambda i: (0, 0)),
            pl.BlockSpec((1, op), lambda i: (0, 0)),
        ],
        out_specs=pl.BlockSpec((tm, op), lambda i: (i, 0)),
        compiler_params=cparams,
    )(a_p, z1, m, w_proj_f, b_proj_f)

    return out_p[:n, :out_ch]
